# baseline (speedup 1.0000x reference)
import jax
import jax.numpy as jnp
from jax import lax
from jax.experimental import pallas as pl
from jax.experimental.pallas import tpu as pltpu
from jax.experimental.pallas import tpu_sc as plsc

B = 8
NMAX = 512
H = 16
F = 768
N_TOTAL = 2048
NEG = -1000000000.0

NC = 1
NS = 16
NW = NC * NS
ROWS_PER_W = B * NMAX // NW
ZERO_ROW = N_TOTAL
NCHUNK = 8
CH = ROWS_PER_W // NCHUNK


def _tc_idx(ids_ref, idx_ref):
    ids = ids_ref[...]
    r = lax.broadcasted_iota(jnp.int32, (NW, ROWS_PER_W), 0) * ROWS_PER_W \
        + lax.broadcasted_iota(jnp.int32, (NW, ROWS_PER_W), 1)
    barr = r // NMAX
    k = r - barr * NMAX
    cnt_arr = jnp.zeros((NW, ROWS_PER_W), jnp.int32)
    cb_arr = jnp.zeros((NW, ROWS_PER_W), jnp.int32)
    for b in range(B):
        cnt = jnp.sum((ids == b).astype(jnp.int32))
        cb = jnp.sum((ids < b).astype(jnp.int32))
        cnt_arr = jnp.where(barr == b, cnt, cnt_arr)
        cb_arr = jnp.where(barr == b, cb, cb_arr)
    idx_ref[...] = jnp.where(k < cnt_arr, cb_arr + k, ZERO_ROW + k)


def _sc_dense(x_hbm, idx_hbm, out_hbm, idx_v, rows_v, semg, semo):
    wid = lax.axis_index("s") * NC + lax.axis_index("c")
    base = wid * ROWS_PER_W
    pltpu.sync_copy(idx_hbm.at[pl.ds(base, ROWS_PER_W)], idx_v)
    for c in range(NCHUNK):
        p = c % 2
        if c >= 2:
            pltpu.make_async_copy(
                rows_v.at[p],
                out_hbm.at[pl.ds(base + (c - 2) * CH, CH)],
                semo.at[p],
            ).wait()
        pltpu.async_copy(
            x_hbm.at[idx_v.at[pl.ds(c * CH, CH)]], rows_v.at[p], semg
        ).wait()
        pltpu.make_async_copy(
            rows_v.at[p], out_hbm.at[pl.ds(base + c * CH, CH)], semo.at[p]
        ).start()
    for c in range(NCHUNK - 2, NCHUNK):
        p = c % 2
        pltpu.make_async_copy(
            rows_v.at[p], out_hbm.at[pl.ds(base + c * CH, CH)], semo.at[p]
        ).wait()


def _tc_mask(ids_ref, mask_hbm, tiles, sem):
    ids = ids_ref[...]
    col = lax.broadcasted_iota(jnp.int32, (NMAX, NMAX), 1)
    for b in range(B):
        cnt = jnp.sum((ids == b).astype(jnp.int32))
        tiles[b] = jnp.where(col >= cnt, NEG, 0.0)
        for h in range(H):
            pltpu.make_async_copy(tiles.at[b], mask_hbm.at[b, h], sem).start()
    for b in range(B):
        for h in range(H):
            pltpu.make_async_copy(tiles.at[b], mask_hbm.at[b, h], sem).wait()


def kernel(x, batch_ids):
    ids2d = batch_ids.astype(jnp.int32).reshape(16, 128)
    x_aug = jnp.concatenate([x, jnp.zeros((NMAX, F), x.dtype)], axis=0)

    idx = pl.pallas_call(
        _tc_idx,
        in_specs=[pl.BlockSpec((16, 128), lambda: (0, 0))],
        out_specs=pl.BlockSpec((NW, ROWS_PER_W), lambda: (0, 0)),
        out_shape=jax.ShapeDtypeStruct((NW, ROWS_PER_W), jnp.int32),
    )(ids2d).reshape(B * NMAX)

    attn_mask = pl.pallas_call(
        _tc_mask,
        in_specs=[pl.BlockSpec((16, 128), lambda: (0, 0))],
        out_specs=pl.BlockSpec(memory_space=pl.ANY),
        out_shape=jax.ShapeDtypeStruct((B, H, NMAX, NMAX), jnp.float32),
        scratch_shapes=[
            pltpu.VMEM((B, NMAX, NMAX), jnp.float32),
            pltpu.SemaphoreType.DMA,
        ],
    )(ids2d)

    dense_flat = pl.kernel(
        _sc_dense,
        out_type=jax.ShapeDtypeStruct((B * NMAX, F), x.dtype),
        mesh=plsc.VectorSubcoreMesh(core_axis_name="c", subcore_axis_name="s",
                                    num_cores=NC),
        scratch_types=[
            pltpu.VMEM((ROWS_PER_W,), jnp.int32),
            pltpu.VMEM((2, CH, F), jnp.float32),
            pltpu.SemaphoreType.DMA,
            pltpu.SemaphoreType.DMA((2,)),
        ],
    )(x_aug, idx)
    dense_x = dense_flat.reshape(B, NMAX, F)
    return dense_x, attn_mask


# speedup vs baseline: 1.6013x; 1.6013x over previous
import jax
import jax.numpy as jnp
from jax import lax
from jax.experimental import pallas as pl
from jax.experimental.pallas import tpu as pltpu

B = 8
NMAX = 512
H = 16
F = 768
N_TOTAL = 2048
NEG = -1000000000.0
CHUNK = 64
WIN = CHUNK + 8


def _kernel(ids_ref, x_hbm, dense_hbm, mask_hbm, xv, tiles, dsc,
            semx, semm, semd):
    pltpu.make_async_copy(x_hbm, xv, semx).start()

    ids = ids_ref[...]
    cnts = [jnp.sum((ids == b).astype(jnp.int32)) for b in range(B)]
    cbs = [jnp.sum((ids < b).astype(jnp.int32)) for b in range(B)]

    col = lax.broadcasted_iota(jnp.int32, (NMAX, NMAX), 1)
    for b in range(B):
        tiles[b] = jnp.where(col >= cnts[b], NEG, 0.0)
        for h in range(H):
            pltpu.make_async_copy(tiles.at[b], mask_hbm.at[b, h], semm).start()

    pltpu.make_async_copy(x_hbm, xv, semx).wait()
    kio = lax.broadcasted_iota(jnp.int32, (CHUNK, 1), 0)
    for b in range(B):
        for j in range(NMAX // CHUNK):
            start = cbs[b] + j * CHUNK
            s = jnp.minimum((start // 8) * 8, N_TOTAL - WIN)
            d = start - s
            win = xv[pl.ds(s, WIN), :]
            rolled = pltpu.roll(win, (WIN - d) % WIN, axis=0)[:CHUNK, :]
            dsc[b, pl.ds(j * CHUNK, CHUNK), :] = jnp.where(
                kio + j * CHUNK < cnts[b], rolled, 0.0
            )
        pltpu.make_async_copy(dsc.at[b], dense_hbm.at[b], semd).start()

    for b in range(B):
        pltpu.make_async_copy(dsc.at[b], dense_hbm.at[b], semd).wait()
    for b in range(B):
        for h in range(H):
            pltpu.make_async_copy(tiles.at[b], mask_hbm.at[b, h], semm).wait()


def kernel(x, batch_ids):
    ids2d = batch_ids.astype(jnp.int32).reshape(16, 128)
    dense_x, attn_mask = pl.pallas_call(
        _kernel,
        in_specs=[
            pl.BlockSpec((16, 128), lambda: (0, 0)),
            pl.BlockSpec(memory_space=pl.ANY),
        ],
        out_specs=[
            pl.BlockSpec(memory_space=pl.ANY),
            pl.BlockSpec(memory_space=pl.ANY),
        ],
        out_shape=[
            jax.ShapeDtypeStruct((B, NMAX, F), x.dtype),
            jax.ShapeDtypeStruct((B, H, NMAX, NMAX), jnp.float32),
        ],
        scratch_shapes=[
            pltpu.VMEM((N_TOTAL, F), jnp.float32),
            pltpu.VMEM((B, NMAX, NMAX), jnp.float32),
            pltpu.VMEM((B, NMAX, F), jnp.float32),
            pltpu.SemaphoreType.DMA,
            pltpu.SemaphoreType.DMA,
            pltpu.SemaphoreType.DMA,
        ],
    )(ids2d, x)
    return dense_x, attn_mask
